# baseline (speedup 1.0000x reference)
import functools

import jax
import jax.numpy as jnp
from jax import lax
from jax.experimental import pallas as pl
from jax.experimental.pallas import tpu as pltpu
from jax.experimental.pallas import tpu_sc as plsc

_CV = 80
_NRINGS = 4
_NDIRS = 8


def _sc_gather(table, sf_flat, nv):
    nch = table.shape[1]
    gw = _NDIRS * nch
    info = plsc.get_sparse_core_info()
    nw = info.num_cores * info.num_subcores
    n_chunks = nv // _CV
    base_per_w = n_chunks // nw
    extra = n_chunks - base_per_w * nw
    trip = _CV * _NRINGS * _NDIRS * 3
    mesh = plsc.VectorSubcoreMesh(core_axis_name="c", subcore_axis_name="s")

    @functools.partial(
        pl.kernel,
        mesh=mesh,
        compiler_params=pltpu.CompilerParams(
            use_tc_tiling_on_sc=False, needs_layout_passes=False),
        out_type=[jax.ShapeDtypeStruct((nv, gw), jnp.float32)
                  for _ in range(_NRINGS)],
        scratch_types=[
            pltpu.VMEM((trip,), jnp.int32),
            pltpu.VMEM((_NRINGS, _NDIRS, _CV), jnp.int32),
            pltpu.VMEM((_NRINGS, _NDIRS, _CV, nch), jnp.float32),
            pltpu.SemaphoreType.DMA((_NRINGS,)),
            pltpu.SemaphoreType.DMA((_NRINGS,)),
        ],
    )
    def gather_kernel(table_hbm, sf_hbm, o0, o1, o2, o3,
                      sfv, idx_v, rows_v, gsem, wsem):
        outs = [o0, o1, o2, o3]
        wid = lax.axis_index("s") * info.num_cores + lax.axis_index("c")
        start_w = wid * base_per_w + lax.min(wid, extra)
        n_w = base_per_w + jnp.where(wid < extra, 1, 0)
        lane96 = lax.iota(jnp.int32, 16) * (_NRINGS * _NDIRS * 3)

        def body(j, carry):
            chunk = start_w + j
            v0 = chunk * _CV
            pltpu.sync_copy(sf_hbm.at[pl.ds(chunk * trip, trip)], sfv)
            for r in range(_NRINGS):
                for dd in range(_NDIRS):
                    off = r * _NDIRS * 3 + dd * 3 + 1
                    for m in range(_CV // 16):
                        word = lane96 + (m * 16 * _NRINGS * _NDIRS * 3 + off)
                        vals = plsc.load_gather(sfv, [word])
                        idx_v[r, dd, pl.ds(16 * m, 16)] = vals

                @pl.when(j > 0)
                def _():
                    for dd in range(_NDIRS):
                        pltpu.make_async_copy(
                            rows_v.at[r, dd],
                            outs[r].at[pl.ds(v0 - _CV, _CV),
                                       pl.ds(dd * nch, nch)],
                            wsem.at[r]).wait()

                for dd in range(_NDIRS):
                    pltpu.async_copy(
                        table_hbm.at[idx_v.at[r, dd]],
                        rows_v.at[r, dd], gsem.at[r])
            for r in range(_NRINGS):
                for dd in range(_NDIRS):
                    pltpu.make_async_copy(
                        table_hbm.at[idx_v.at[r, dd]],
                        rows_v.at[r, dd], gsem.at[r]).wait()
                for dd in range(_NDIRS):
                    pltpu.async_copy(
                        rows_v.at[r, dd],
                        outs[r].at[pl.ds(v0, _CV), pl.ds(dd * nch, nch)],
                        wsem.at[r])
            return carry

        lax.fori_loop(0, n_w, body, 0)
        vlast = (start_w + n_w - 1) * _CV
        for r in range(_NRINGS):
            for dd in range(_NDIRS):
                pltpu.make_async_copy(
                    rows_v.at[r, dd],
                    outs[r].at[pl.ds(vlast, _CV), pl.ds(dd * nch, nch)],
                    wsem.at[r]).wait()

    return gather_kernel(table, sf_flat)


def _tc_conv(gs, ws, y2, w2, b2, blk):
    nv = y2.shape[0]
    ncols = w2.shape[1]
    nf = ncols // 8

    def body(g0, g1, g2, g3, w0, w1, w2r, w3, y_ref, wc, b_ref, o_ref):
        gr = (g0, g1, g2, g3)
        wr = (w0, w1, w2r, w3)
        acc = jnp.dot(y_ref[...], wc[...], preferred_element_type=jnp.float32)
        for r in range(4):
            acc = acc + jnp.dot(gr[r][...], wr[r][...],
                                preferred_element_type=jnp.float32)
        acc = acc + b_ref[...]
        acc = jnp.maximum(acc, 0.0)
        m = jnp.maximum(acc[:, 0:4 * nf], acc[:, 4 * nf:8 * nf])
        m = jnp.maximum(m[:, 0:2 * nf], m[:, 2 * nf:4 * nf])
        o_ref[...] = jnp.maximum(m[:, 0:nf], m[:, nf:2 * nf])

    g_spec = pl.BlockSpec((blk, ncols), lambda i: (i, 0))
    w_spec = pl.BlockSpec((ncols, ncols), lambda i: (0, 0))
    return pl.pallas_call(
        body,
        grid=(nv // blk,),
        in_specs=[g_spec] * 4 + [w_spec] * 4 + [
            pl.BlockSpec((blk, y2.shape[1]), lambda i: (i, 0)),
            pl.BlockSpec((y2.shape[1], ncols), lambda i: (0, 0)),
            pl.BlockSpec((1, ncols), lambda i: (0, 0)),
        ],
        out_specs=pl.BlockSpec((blk, nf), lambda i: (i, 0)),
        out_shape=jax.ShapeDtypeStruct((nv, nf), jnp.float32),
    )(*gs, *ws, y2, w2, b2)


def kernel(y, sync_field, kernel, center_kernel, bias):
    nb, nv, nch = y.shape
    nrings, ndirs, _, nf = kernel.shape

    table = y.reshape(nb * nv, nch)
    sf_flat = sync_field.reshape(-1)

    gs = _sc_gather(table, sf_flat, nb * nv)

    dd = jnp.arange(ndirs)
    rot = (dd[:, None] - dd[None, :]) % ndirs
    Krot = kernel[:, rot, :, :]
    W = jnp.transpose(Krot, (0, 1, 3, 2, 4)).reshape(
        nrings, ndirs * nch, ndirs * nf)
    ws = [W[r] for r in range(nrings)]
    w2 = jnp.tile(center_kernel, (1, ndirs))
    b2 = jnp.tile(bias, (ndirs,))[None, :]

    out = _tc_conv(gs, ws, table, w2, b2, blk=2000)
    return out.reshape(nb, nv, nf)
